# Initial kernel scaffold; baseline (speedup 1.0000x reference)
import jax
import jax.numpy as jnp
from jax.experimental import pallas as pl


def kernel(bbox, x_emb, y_emb):
    raise NotImplementedError("write your pallas kernel here")


# speedup vs baseline: 1.0369x; 1.0369x over previous
import functools

import jax
import jax.numpy as jnp
from jax import lax
from jax.experimental import pallas as pl
from jax.experimental.pallas import tpu as pltpu
from jax.experimental.pallas import tpu_sc as plsc

MAX2D = 501
D = 1024
TOKENS = 32768
NW = 32
TPW = TOKENS // NW
CHUNK = 16
NCHUNK = TPW // CHUNK
LANES = 16


def _sc_body(bbox_hbm, table_hbm, out_hbm, bbox_v, idx_v, rows_v, out_v, sem):
    wid = lax.axis_index("s") * 2 + lax.axis_index("c")
    base = wid * (TPW * 4)

    pltpu.sync_copy(bbox_hbm.at[pl.ds(base, TPW * 4)], bbox_v)

    lane = lax.iota(jnp.int32, LANES)
    y_off = (lane & 1) * MAX2D

    def compute_idx(c, _):
        for j in range(4):
            p = c * (CHUNK * 4) + j * LANES
            v = bbox_v[pl.ds(p, LANES)]
            v = jnp.minimum(jnp.maximum(v, 0.0), 1.0)
            idx = (v * float(MAX2D - 1)).astype(jnp.int32) + y_off
            idx_v[c, pl.ds(j * LANES, LANES)] = idx
        return 0

    lax.fori_loop(0, NCHUNK, compute_idx, 0)

    def do_chunk(c, _):
        pltpu.async_copy(table_hbm.at[idx_v.at[c]], rows_v, sem).wait()
        def sum_token(t, _):
            for k in range(D // LANES):
                s = pl.ds(k * LANES, LANES)
                acc = rows_v[t * 4, s] + rows_v[t * 4 + 1, s]
                acc = acc + rows_v[t * 4 + 2, s]
                out_v[t, s] = acc + rows_v[t * 4 + 3, s]
            return 0

        lax.fori_loop(0, CHUNK, sum_token, 0)
        pltpu.async_copy(
            out_v, out_hbm.at[pl.ds(wid * TPW + c * CHUNK, CHUNK)], sem
        ).wait()
        return 0

    lax.fori_loop(0, NCHUNK, do_chunk, 0)


@jax.jit
def _cell_embed(bbox_flat, table):
    mesh = plsc.VectorSubcoreMesh(
        core_axis_name="c", subcore_axis_name="s", num_cores=2, num_subcores=16
    )
    return pl.kernel(
        _sc_body,
        out_type=jax.ShapeDtypeStruct((TOKENS, D), jnp.float32),
        mesh=mesh,
        scratch_types=[
            pltpu.VMEM((TPW * 4,), jnp.float32),
            pltpu.VMEM((NCHUNK, CHUNK * 4), jnp.int32),
            pltpu.VMEM((CHUNK * 4, D), jnp.float32),
            pltpu.VMEM((CHUNK, D), jnp.float32),
            pltpu.SemaphoreType.DMA,
        ],
    )(bbox_flat, table)


def kernel(bbox, x_emb, y_emb):
    b, s, _ = bbox.shape
    table = jnp.concatenate([x_emb, y_emb], axis=0)
    out = _cell_embed(bbox.reshape(-1), table)
    return out.reshape(b, s, D)


# speedup vs baseline: 2.7133x; 2.6167x over previous
import jax
import jax.numpy as jnp
from jax import lax
from jax.experimental import pallas as pl
from jax.experimental.pallas import tpu as pltpu
from jax.experimental.pallas import tpu_sc as plsc

MAX2D = 501
D = 1024
TOKENS = 32768
NW = 32
TPW = TOKENS // NW
CHUNK = 8
NCHUNK = TPW // CHUNK
LANES = 16
NTRIP = (NCHUNK - 2) // 3


def _sc_body(bbox_hbm, table_hbm, out_hbm, idx_v, r0, r1, r2,
             g0, g1, g2, o0, o1, o2):
    wid = lax.axis_index("s") * 2 + lax.axis_index("c")
    bufs = (r0, r1, r2)
    gsems = (g0, g1, g2)
    osems = (o0, o1, o2)

    pltpu.sync_copy(bbox_hbm.at[wid], r0.at[pl.ds(0, 4)])

    for j in range(4):
        off = 0 if j % 2 == 0 else MAX2D

        def compute_idx(i, _, j=j, off=off):
            v = r0[j, pl.ds(i * LANES, LANES)]
            v = jnp.minimum(jnp.maximum(v, 0.0), 1.0)
            idx = (v * float(MAX2D - 1)).astype(jnp.int32) + off
            idx_v[pl.ds(j * TPW + i * LANES, LANES)] = idx
            return 0

        lax.fori_loop(0, TPW // LANES, compute_idx, 0)

    def gather(c, par):
        for j in range(4):
            pltpu.async_copy(
                table_hbm.at[idx_v.at[pl.ds(j * TPW + c * CHUNK, CHUNK)]],
                bufs[par].at[pl.ds(j * CHUNK, CHUNK)], gsems[par],
            )

    def fold_and_out(c, par):
        buf = bufs[par]
        pltpu.make_async_copy(
            table_hbm.at[pl.ds(0, 4 * CHUNK)], buf, gsems[par]
        ).wait()

        def fold(t, _):
            for k in range(D // LANES):
                s = pl.ds(k * LANES, LANES)
                v = buf[CHUNK + t, s] + buf[2 * CHUNK + t, s]
                v = v + buf[3 * CHUNK + t, s]
                plsc.addupdate(buf.at[t, s], v)
            return 0

        lax.fori_loop(0, CHUNK, fold, 0)
        pltpu.async_copy(
            buf.at[pl.ds(0, CHUNK)],
            out_hbm.at[pl.ds(wid * TPW + c * CHUNK, CHUNK)],
            osems[par],
        )

    def drain_out(par):
        pltpu.make_async_copy(
            bufs[par].at[pl.ds(0, CHUNK)], out_hbm.at[pl.ds(0, CHUNK)],
            osems[par],
        ).wait()

    gather(0, 0)

    def do_triple(c3, _):
        for par in range(3):
            c = c3 * 3 + par
            parn = (par + 1) % 3
            if par == 2:
                drain_out(parn)
            else:
                @pl.when(c3 > 0)
                def _():
                    drain_out(parn)
            gather(c + 1, parn)
            fold_and_out(c, par)
        return 0

    lax.fori_loop(0, NTRIP, do_triple, 0)

    c = NTRIP * 3
    drain_out(1)
    gather(c + 1, 1)
    fold_and_out(c, 0)
    fold_and_out(c + 1, 1)
    drain_out(2)
    drain_out(0)
    drain_out(1)


@jax.jit
def _cell_embed(bbox_blocks, table):
    mesh = plsc.VectorSubcoreMesh(
        core_axis_name="c", subcore_axis_name="s", num_cores=2, num_subcores=16
    )
    return pl.kernel(
        _sc_body,
        out_type=jax.ShapeDtypeStruct((TOKENS, D), jnp.float32),
        mesh=mesh,
        scratch_types=[
            pltpu.VMEM((TPW * 4,), jnp.int32),
            pltpu.VMEM((4 * CHUNK, D), jnp.float32),
            pltpu.VMEM((4 * CHUNK, D), jnp.float32),
            pltpu.VMEM((4 * CHUNK, D), jnp.float32),
            pltpu.SemaphoreType.DMA,
            pltpu.SemaphoreType.DMA,
            pltpu.SemaphoreType.DMA,
            pltpu.SemaphoreType.DMA,
            pltpu.SemaphoreType.DMA,
            pltpu.SemaphoreType.DMA,
        ],
    )(bbox_blocks, table)


def kernel(bbox, x_emb, y_emb):
    b, s, _ = bbox.shape
    table = jnp.concatenate([x_emb, y_emb], axis=0)
    bbox_blocks = (
        bbox.reshape(-1, 4).T.reshape(4, NW, TPW).transpose(1, 0, 2)
    )
    out = _cell_embed(bbox_blocks, table)
    return out.reshape(b, s, D)
